# baseline (speedup 1.0000x reference)
import jax
import jax.numpy as jnp
import numpy as np
from jax import lax
from jax.experimental import pallas as pl
from jax.experimental.pallas import tpu as pltpu

H = 12
TOPK = 256
INT_MIN32 = np.int32(-2147483648)
NEG_BIG = np.float32(-3.0e38)


def _attn_body(x_ref, wq_ref, bq_ref, wkv_ref, bkv_ref, e_ref, g_ref, w2_ref,
               kv_ref, T, N, C, HD, scale):
    TH = T * H
    NFp = N

    xb = x_ref[0]
    kv = lax.dot_general(xb, wkv_ref[...], (((1,), (1,)), ((), ())),
                         preferred_element_type=jnp.float32)
    kv_ref[...] = kv + bkv_ref[...]
    k = kv_ref[:, :C]
    v = kv_ref[:, C:]

    qrows = []
    for t in range(T):
        qt = lax.dot_general(xb[t:t + 1, :], wq_ref[t],
                             (((1,), (1,)), ((), ())),
                             preferred_element_type=jnp.float32)
        qrows.append(qt + bq_ref[t:t + 1, :])
    q = jnp.concatenate(qrows, axis=0)

    e = e_ref[...]
    qp = (q[:, None, :] * e[None, :, :]).reshape(TH, C)

    a = lax.dot_general(qp, k, (((1,), (1,)), ((), ())),
                        preferred_element_type=jnp.float32) * scale
    col = lax.broadcasted_iota(jnp.int32, (TH, NFp), 1)
    a = jnp.where(col < T, NEG_BIG, a)

    bits = lax.bitcast_convert_type(a, jnp.int32)
    ikey = bits ^ (np.int32(0x7FFFFFFF) & (bits >> 31))
    tu = jnp.zeros((TH, 1), jnp.int32)
    for j in range(31, -1, -1):
        cand = tu | np.int32(1 << j) if j < 31 else tu | INT_MIN32
        thr_s = cand ^ INT_MIN32
        cnt = jnp.sum((ikey >= thr_s).astype(jnp.int32), axis=1, keepdims=True)
        tu = jnp.where(cnt >= TOPK, cand, tu)
    thr_s = tu ^ INT_MIN32
    mask = ikey >= thr_s

    rowmax = jnp.max(a, axis=1, keepdims=True)
    ex = jnp.where(mask, jnp.exp(a - rowmax), 0.0)
    denom = jnp.sum(ex, axis=1, keepdims=True)
    p_row = ex / denom

    at3 = lax.dot_general(p_row, v, (((1,), (0,)), ((), ())),
                          preferred_element_type=jnp.float32)
    at = jnp.sum(at3.reshape(T, H, C) * e[None, :, :], axis=1)

    g_ref[0] = xb
    g_ref[0, 0:T, :] = at

    i36 = (lax.broadcasted_iota(jnp.int32, (TH, TH), 0)
           == lax.broadcasted_iota(jnp.int32, (TH, TH), 1)).astype(jnp.float32)
    stats = jnp.concatenate([rowmax, denom], axis=1)
    stats_t = lax.dot_general(stats, i36, (((0,), (0,)), ((), ())),
                              preferred_element_type=jnp.float32)
    max_r = stats_t[0:1, :]
    den_r = stats_t[1:2, :]
    mask_t = lax.dot_general(mask.astype(jnp.float32), i36,
                             (((0,), (0,)), ((), ())),
                             preferred_element_type=jnp.float32)

    a2 = lax.dot_general(k, qp, (((1,), (1,)), ((), ())),
                         preferred_element_type=jnp.float32) * scale
    row = lax.broadcasted_iota(jnp.int32, (NFp, TH), 0)
    lane = lax.broadcasted_iota(jnp.int32, (NFp, TH), 1)
    a2 = jnp.where(row < T, NEG_BIG, a2)
    p2 = jnp.where(mask_t > 0.5, jnp.exp(a2 - max_r), 0.0) / den_r
    onehot = ((lane // H) == row).astype(jnp.float32)
    w2_ref[0] = jnp.where(row < T, onehot, p2)


def _out_body(g_ref, w2_ref, e_ref, wexp_ref, out_ref, T):
    g = g_ref[0]
    w2 = w2_ref[0]
    acc = None
    for t in range(T):
        wt = w2[:, t * H:(t + 1) * H]
        wx = lax.dot_general(wt, e_ref[...], (((1,), (0,)), ((), ())),
                             preferred_element_type=jnp.float32)
        contrib = lax.dot_general(g * wx, wexp_ref[t],
                                  (((1,), (1,)), ((), ())),
                                  preferred_element_type=jnp.float32)
        acc = contrib if acc is None else acc + contrib
    out_ref[0] = acc


def kernel(x, Wq, bq, Wkv, bkv, Wexp):
    B, N, C = x.shape
    T = Wq.shape[0]
    HD = C // H
    scale = HD ** (-0.5)
    C2 = 2 * C
    TH = T * H

    E = (jnp.arange(C, dtype=jnp.int32)[None, :] // HD
         == jnp.arange(H, dtype=jnp.int32)[:, None]).astype(jnp.float32)

    attn_fn = lambda xr, wq, bqr, wkv, bkvr, er, gr, w2r, kvr: _attn_body(
        xr, wq, bqr, wkv, bkvr, er, gr, w2r, kvr, T, N, C, HD, scale)
    g, w2 = pl.pallas_call(
        attn_fn,
        grid=(B,),
        in_specs=[
            pl.BlockSpec((1, N, C), lambda b: (b, 0, 0)),
            pl.BlockSpec((T, C, C), lambda b: (0, 0, 0)),
            pl.BlockSpec((T, C), lambda b: (0, 0)),
            pl.BlockSpec((C2, C), lambda b: (0, 0)),
            pl.BlockSpec((1, C2), lambda b: (0, 0)),
            pl.BlockSpec((H, C), lambda b: (0, 0)),
        ],
        out_specs=[
            pl.BlockSpec((1, N, C), lambda b: (b, 0, 0)),
            pl.BlockSpec((1, N, TH), lambda b: (b, 0, 0)),
        ],
        out_shape=[
            jax.ShapeDtypeStruct((B, N, C), jnp.float32),
            jax.ShapeDtypeStruct((B, N, TH), jnp.float32),
        ],
        scratch_shapes=[pltpu.VMEM((N, C2), jnp.float32)],
    )(x, Wq, bq, Wkv, bkv.reshape(1, C2), E)

    BS = 296
    NB = -(-N // BS)
    out_fn = lambda gr, w2r, er, wer, outr: _out_body(gr, w2r, er, wer, outr, T)
    out = pl.pallas_call(
        out_fn,
        grid=(B, NB),
        in_specs=[
            pl.BlockSpec((1, BS, C), lambda b, n: (b, n, 0)),
            pl.BlockSpec((1, BS, TH), lambda b, n: (b, n, 0)),
            pl.BlockSpec((H, C), lambda b, n: (0, 0)),
            pl.BlockSpec((T, C, C), lambda b, n: (0, 0, 0)),
        ],
        out_specs=pl.BlockSpec((1, BS, C), lambda b, n: (b, n, 0)),
        out_shape=jax.ShapeDtypeStruct((B, N, C), jnp.float32),
    )(g, w2, E, Wexp)

    return out


# speedup vs baseline: 1.1042x; 1.1042x over previous
import jax
import jax.numpy as jnp
import numpy as np
from jax import lax
from jax.experimental import pallas as pl
from jax.experimental.pallas import tpu as pltpu

H = 12
TOPK = 256
INT_MIN32 = np.int32(-2147483648)
NEG_BIG = np.float32(-3.0e38)


def _attn_body(x_ref, wq_ref, bq_ref, wkv_ref, bkv_ref, e_ref, at_ref, w2_ref,
               T, N, C, HD, scale):
    TH = T * H
    xb = x_ref[0]

    qrows = []
    for t in range(T):
        qt = lax.dot_general(xb[t:t + 1, :], wq_ref[t],
                             (((1,), (1,)), ((), ())),
                             preferred_element_type=jnp.float32)
        qrows.append(qt + bq_ref[t:t + 1, :])
    q = jnp.concatenate(qrows, axis=0)

    e = e_ref[...]
    qp = (q[:, None, :] * e[None, :, :]).reshape(TH, C)

    u = lax.dot_general(qp, wkv_ref[0:C, :], (((1,), (0,)), ((), ())),
                        preferred_element_type=jnp.float32)
    bias_k = lax.dot_general(qp, bkv_ref[0:1, 0:C], (((1,), (1,)), ((), ())),
                             preferred_element_type=jnp.float32)
    a = (lax.dot_general(u, xb, (((1,), (1,)), ((), ())),
                         preferred_element_type=jnp.float32)
         + bias_k) * scale
    col = lax.broadcasted_iota(jnp.int32, (TH, N), 1)
    a = jnp.where(col < T, NEG_BIG, a)

    bits = lax.bitcast_convert_type(a, jnp.int32)
    ikey = bits ^ (np.int32(0x7FFFFFFF) & (bits >> 31))
    tu = jnp.zeros((TH, 1), jnp.int32)
    for j in range(31, -1, -1):
        cand = tu | np.int32(1 << j) if j < 31 else tu | INT_MIN32
        thr_s = cand ^ INT_MIN32
        cnt = jnp.sum((ikey >= thr_s).astype(jnp.int32), axis=1, keepdims=True)
        tu = jnp.where(cnt >= TOPK, cand, tu)
    thr_s = tu ^ INT_MIN32
    mask = ikey >= thr_s

    rowmax = jnp.max(a, axis=1, keepdims=True)
    ex = jnp.where(mask, jnp.exp(a - rowmax), 0.0)
    denom = jnp.sum(ex, axis=1, keepdims=True)
    p_row = ex / denom

    px = lax.dot_general(p_row, xb, (((1,), (0,)), ((), ())),
                         preferred_element_type=jnp.float32)
    at3 = (lax.dot_general(px, wkv_ref[C:2 * C, :], (((1,), (1,)), ((), ())),
                           preferred_element_type=jnp.float32)
           + bkv_ref[0:1, C:2 * C])
    at_ref[0] = jnp.sum(at3.reshape(T, H, C) * e[None, :, :], axis=1)

    i36 = (lax.broadcasted_iota(jnp.int32, (TH, TH), 0)
           == lax.broadcasted_iota(jnp.int32, (TH, TH), 1)).astype(jnp.float32)
    stats = jnp.concatenate([rowmax, denom], axis=1)
    stats_t = lax.dot_general(stats, i36, (((0,), (0,)), ((), ())),
                              preferred_element_type=jnp.float32)
    max_r = stats_t[0:1, :]
    den_r = stats_t[1:2, :]
    mask_t = lax.dot_general(mask.astype(jnp.float32), i36,
                             (((0,), (0,)), ((), ())),
                             preferred_element_type=jnp.float32)

    bias_t = lax.dot_general(bias_k, i36, (((0,), (0,)), ((), ())),
                             preferred_element_type=jnp.float32)
    a2 = (lax.dot_general(xb, u, (((1,), (1,)), ((), ())),
                          preferred_element_type=jnp.float32)
          + bias_t) * scale
    row = lax.broadcasted_iota(jnp.int32, (N, TH), 0)
    lane = lax.broadcasted_iota(jnp.int32, (N, TH), 1)
    a2 = jnp.where(row < T, NEG_BIG, a2)
    p2 = jnp.where(mask_t > 0.5, jnp.exp(a2 - max_r), 0.0) / den_r
    onehot = ((lane // H) == row).astype(jnp.float32)
    w2_ref[0] = jnp.where(row < T, onehot, p2)


def _out_body(x_ref, at_ref, w2_ref, e_ref, wexp_ref, out_ref, g_ref, T):
    n = pl.program_id(1)
    g_ref[...] = x_ref[0]

    @pl.when(n == 0)
    def _():
        g_ref[0:T, :] = at_ref[0]

    g = g_ref[...]
    w2 = w2_ref[0]
    acc = None
    for t in range(T):
        wt = w2[:, t * H:(t + 1) * H]
        wx = lax.dot_general(wt, e_ref[...], (((1,), (0,)), ((), ())),
                             preferred_element_type=jnp.float32)
        contrib = lax.dot_general(g * wx, wexp_ref[t],
                                  (((1,), (1,)), ((), ())),
                                  preferred_element_type=jnp.float32)
        acc = contrib if acc is None else acc + contrib
    out_ref[0] = acc


def kernel(x, Wq, bq, Wkv, bkv, Wexp):
    B, N, C = x.shape
    T = Wq.shape[0]
    HD = C // H
    scale = HD ** (-0.5)
    C2 = 2 * C
    TH = T * H

    E = (jnp.arange(C, dtype=jnp.int32)[None, :] // HD
         == jnp.arange(H, dtype=jnp.int32)[:, None]).astype(jnp.float32)

    attn_fn = lambda xr, wq, bqr, wkv, bkvr, er, atr, w2r: _attn_body(
        xr, wq, bqr, wkv, bkvr, er, atr, w2r, T, N, C, HD, scale)
    at, w2 = pl.pallas_call(
        attn_fn,
        grid=(B,),
        in_specs=[
            pl.BlockSpec((1, N, C), lambda b: (b, 0, 0)),
            pl.BlockSpec((T, C, C), lambda b: (0, 0, 0)),
            pl.BlockSpec((T, C), lambda b: (0, 0)),
            pl.BlockSpec((C2, C), lambda b: (0, 0)),
            pl.BlockSpec((1, C2), lambda b: (0, 0)),
            pl.BlockSpec((H, C), lambda b: (0, 0)),
        ],
        out_specs=[
            pl.BlockSpec((1, T, C), lambda b: (b, 0, 0)),
            pl.BlockSpec((1, N, TH), lambda b: (b, 0, 0)),
        ],
        out_shape=[
            jax.ShapeDtypeStruct((B, T, C), jnp.float32),
            jax.ShapeDtypeStruct((B, N, TH), jnp.float32),
        ],
    )(x, Wq, bq, Wkv, bkv.reshape(1, C2), E)

    BS = 296
    NB = -(-N // BS)
    out_fn = lambda xr, atr, w2r, er, wer, outr, gr: _out_body(
        xr, atr, w2r, er, wer, outr, gr, T)
    out = pl.pallas_call(
        out_fn,
        grid=(B, NB),
        in_specs=[
            pl.BlockSpec((1, BS, C), lambda b, n: (b, n, 0)),
            pl.BlockSpec((1, T, C), lambda b, n: (b, 0, 0)),
            pl.BlockSpec((1, BS, TH), lambda b, n: (b, n, 0)),
            pl.BlockSpec((H, C), lambda b, n: (0, 0)),
            pl.BlockSpec((T, C, C), lambda b, n: (0, 0, 0)),
        ],
        out_specs=pl.BlockSpec((1, BS, C), lambda b, n: (b, n, 0)),
        out_shape=jax.ShapeDtypeStruct((B, N, C), jnp.float32),
        scratch_shapes=[pltpu.VMEM((BS, C), jnp.float32)],
    )(x, at, w2, E, Wexp)

    return out
